# initial kernel scaffold (unmeasured)
import jax
import jax.numpy as jnp
from jax import lax
from jax.experimental import pallas as pl
from jax.experimental.pallas import tpu as pltpu

N_DEV = 8


def kernel(x, Win0, Wout0, Win1, Wout1, Win2, Wout2):
    b, d_model = x.shape
    rows = b // N_DEV

    def body(x_ref, win0_ref, wout0_ref, win1_ref, wout1_ref, win2_ref,
             wout2_ref, out_ref, part_ref, rs_ref, ag_ref,
             rs_send_sems, rs_recv_sems, ag_send_sems, ag_recv_sems):
        my = lax.axis_index("i")

        def layer_partial(x_f32, win_ref, wout_ref):
            xb = x_f32.astype(jnp.bfloat16)
            w_in = win_ref[...].astype(jnp.bfloat16)
            w_out = wout_ref[...].astype(jnp.bfloat16)
            h = jnp.dot(xb, w_in, preferred_element_type=jnp.float32)
            h = jnp.maximum(h, 0.0).astype(jnp.bfloat16)
            return jnp.dot(h, w_out, preferred_element_type=jnp.float32)

        def reduce_scatter():
            sends = []
            for off in range(1, N_DEV):
                d = lax.rem(my + off, N_DEV)
                rdma = pltpu.make_async_remote_copy(
                    src_ref=part_ref.at[pl.ds(d * rows, rows), :],
                    dst_ref=rs_ref.at[my],
                    send_sem=rs_send_sems.at[d],
                    recv_sem=rs_recv_sems.at[my],
                    device_id=(d,),
                    device_id_type=pl.DeviceIdType.MESH,
                )
                rdma.start()
                sends.append(rdma)
            rs_ref[my] = part_ref[pl.ds(my * rows, rows), :]
            for off in range(1, N_DEV):
                s = lax.rem(my + off, N_DEV)
                recv = pltpu.make_async_remote_copy(
                    src_ref=part_ref.at[pl.ds(s * rows, rows), :],
                    dst_ref=rs_ref.at[s],
                    send_sem=rs_send_sems.at[s],
                    recv_sem=rs_recv_sems.at[s],
                    device_id=(s,),
                    device_id_type=pl.DeviceIdType.MESH,
                )
                recv.wait_recv()
            for rdma in sends:
                rdma.wait_send()
            return jnp.sum(rs_ref[...], axis=0)

        def all_gather(reduced):
            ag_ref[my] = reduced
            sends = []
            for off in range(1, N_DEV):
                d = lax.rem(my + off, N_DEV)
                rdma = pltpu.make_async_remote_copy(
                    src_ref=ag_ref.at[my],
                    dst_ref=ag_ref.at[my],
                    send_sem=ag_send_sems.at[d],
                    recv_sem=ag_recv_sems.at[my],
                    device_id=(d,),
                    device_id_type=pl.DeviceIdType.MESH,
                )
                rdma.start()
                sends.append(rdma)
            for off in range(1, N_DEV):
                s = lax.rem(my + off, N_DEV)
                recv = pltpu.make_async_remote_copy(
                    src_ref=ag_ref.at[s],
                    dst_ref=ag_ref.at[s],
                    send_sem=ag_send_sems.at[s],
                    recv_sem=ag_recv_sems.at[s],
                    device_id=(s,),
                    device_id_type=pl.DeviceIdType.MESH,
                )
                recv.wait_recv()
            for rdma in sends:
                rdma.wait_send()
            return ag_ref[...].reshape(b, d_model)

        x_cur = x_ref[...]
        for lidx, (win_ref, wout_ref) in enumerate(
            [(win0_ref, wout0_ref), (win1_ref, wout1_ref),
             (win2_ref, wout2_ref)]
        ):
            part_ref[...] = layer_partial(x_cur, win_ref, wout_ref)
            reduced = reduce_scatter()
            if lidx < 2:
                x_cur = all_gather(reduced)
            else:
                out_ref[...] = reduced

    out_shape = jax.ShapeDtypeStruct((rows, d_model), jnp.float32)
    vmem = pl.BlockSpec(memory_space=pltpu.VMEM)
    return pl.pallas_call(
        body,
        out_shape=out_shape,
        in_specs=[vmem] * 7,
        out_specs=vmem,
        scratch_shapes=[
            pltpu.VMEM((b, d_model), jnp.float32),
            pltpu.VMEM((N_DEV, rows, d_model), jnp.float32),
            pltpu.VMEM((N_DEV, rows, d_model), jnp.float32),
            pltpu.SemaphoreType.DMA((N_DEV,)),
            pltpu.SemaphoreType.DMA((N_DEV,)),
            pltpu.SemaphoreType.DMA((N_DEV,)),
            pltpu.SemaphoreType.DMA((N_DEV,)),
        ],
        compiler_params=pltpu.CompilerParams(collective_id=0),
    )(x, Win0, Wout0, Win1, Wout1, Win2, Wout2)


# baseline (device time: 35838 ns/iter reference)
import jax
import jax.numpy as jnp
from jax import lax
from jax.experimental import pallas as pl
from jax.experimental.pallas import tpu as pltpu

N_DEV = 8


def kernel(x, Win0, Wout0, Win1, Wout1, Win2, Wout2):
    b, d_model = x.shape
    rows = b // N_DEV

    def body(x_ref, win0_ref, wout0_ref, win1_ref, wout1_ref, win2_ref,
             wout2_ref, out_ref, part_ref, rs_ref, ag_ref,
             rs_send_sems, rs_recv_sems, ag_send_sems, ag_recv_sems):
        my = lax.axis_index("i")

        def layer_partial(x_f32, win_ref, wout_ref):
            xb = x_f32.astype(jnp.bfloat16)
            w_in = win_ref[...].astype(jnp.bfloat16)
            w_out = wout_ref[...].astype(jnp.bfloat16)
            h = jnp.dot(xb, w_in, preferred_element_type=jnp.float32)
            h = jnp.maximum(h, 0.0).astype(jnp.bfloat16)
            return jnp.dot(h, w_out, preferred_element_type=jnp.float32)

        def reduce_scatter():
            sends = []
            for off in range(1, N_DEV):
                d = lax.rem(my + off, N_DEV)
                rdma = pltpu.make_async_remote_copy(
                    src_ref=part_ref.at[pl.ds(d * rows, rows), :],
                    dst_ref=rs_ref.at[my],
                    send_sem=rs_send_sems.at[d],
                    recv_sem=rs_recv_sems.at[my],
                    device_id=(d,),
                    device_id_type=pl.DeviceIdType.MESH,
                )
                rdma.start()
                sends.append(rdma)
            rs_ref[my] = part_ref[pl.ds(my * rows, rows), :]
            for off in range(1, N_DEV):
                s = lax.rem(my + off, N_DEV)
                recv = pltpu.make_async_remote_copy(
                    src_ref=part_ref.at[pl.ds(s * rows, rows), :],
                    dst_ref=rs_ref.at[s],
                    send_sem=rs_send_sems.at[s],
                    recv_sem=rs_recv_sems.at[s],
                    device_id=(s,),
                    device_id_type=pl.DeviceIdType.MESH,
                )
                recv.wait_recv()
            for rdma in sends:
                rdma.wait_send()
            return jnp.sum(rs_ref[...], axis=0)

        def all_gather(reduced):
            ag_ref[my] = reduced
            sends = []
            for off in range(1, N_DEV):
                d = lax.rem(my + off, N_DEV)
                rdma = pltpu.make_async_remote_copy(
                    src_ref=ag_ref.at[my],
                    dst_ref=ag_ref.at[my],
                    send_sem=ag_send_sems.at[d],
                    recv_sem=ag_recv_sems.at[my],
                    device_id=(d,),
                    device_id_type=pl.DeviceIdType.MESH,
                )
                rdma.start()
                sends.append(rdma)
            for off in range(1, N_DEV):
                s = lax.rem(my + off, N_DEV)
                recv = pltpu.make_async_remote_copy(
                    src_ref=ag_ref.at[s],
                    dst_ref=ag_ref.at[s],
                    send_sem=ag_send_sems.at[s],
                    recv_sem=ag_recv_sems.at[s],
                    device_id=(s,),
                    device_id_type=pl.DeviceIdType.MESH,
                )
                recv.wait_recv()
            for rdma in sends:
                rdma.wait_send()
            return ag_ref[...].reshape(b, d_model)

        x_cur = x_ref[...]
        for lidx, (win_ref, wout_ref) in enumerate(
            [(win0_ref, wout0_ref), (win1_ref, wout1_ref),
             (win2_ref, wout2_ref)]
        ):
            part_ref[...] = layer_partial(x_cur, win_ref, wout_ref)
            reduced = reduce_scatter()
            if lidx < 2:
                x_cur = all_gather(reduced)
            else:
                out_ref[...] = reduced

    out_shape = jax.ShapeDtypeStruct((rows, d_model), jnp.float32)
    vmem = pl.BlockSpec(memory_space=pltpu.VMEM)
    return pl.pallas_call(
        body,
        out_shape=out_shape,
        in_specs=[vmem] * 7,
        out_specs=vmem,
        scratch_shapes=[
            pltpu.VMEM((b, d_model), jnp.float32),
            pltpu.VMEM((N_DEV, rows, d_model), jnp.float32),
            pltpu.VMEM((N_DEV, rows, d_model), jnp.float32),
            pltpu.SemaphoreType.DMA((N_DEV,)),
            pltpu.SemaphoreType.DMA((N_DEV,)),
            pltpu.SemaphoreType.DMA((N_DEV,)),
            pltpu.SemaphoreType.DMA((N_DEV,)),
        ],
    )(x, Win0, Wout0, Win1, Wout1, Win2, Wout2)


# device time: 33918 ns/iter; 1.0566x vs baseline; 1.0566x over previous
import jax
import jax.numpy as jnp
from jax import lax
from jax.experimental import pallas as pl
from jax.experimental.pallas import tpu as pltpu

N_DEV = 8


def kernel(x, Win0, Wout0, Win1, Wout1, Win2, Wout2):
    b, d_model = x.shape
    rows = b // N_DEV

    def body(x_ref, win0_ref, wout0_ref, win1_ref, wout1_ref, win2_ref,
             wout2_ref, out_ref, part_ref, rs_ref, ag_ref,
             rs_send_sems, rs_recv_sems, ag_send_sems, ag_recv_sems):
        my = lax.axis_index("i")

        def layer_partial(x_bf16, win_ref, wout_ref):
            w_in = win_ref[...].astype(jnp.bfloat16)
            w_out = wout_ref[...].astype(jnp.bfloat16)
            h = jnp.dot(x_bf16, w_in, preferred_element_type=jnp.float32)
            h = jnp.maximum(h, 0.0).astype(jnp.bfloat16)
            return jnp.dot(h, w_out, preferred_element_type=jnp.float32)

        def rs_send(chunk_idx):
            rdma = pltpu.make_async_remote_copy(
                src_ref=part_ref.at[pl.ds(chunk_idx * rows, rows), :],
                dst_ref=rs_ref.at[my],
                send_sem=rs_send_sems.at[chunk_idx],
                recv_sem=rs_recv_sems.at[my],
                device_id=(chunk_idx,),
                device_id_type=pl.DeviceIdType.MESH,
            )
            rdma.start()
            return rdma

        def rs_finish(sends):
            for off in range(1, N_DEV):
                s = lax.rem(my + off, N_DEV)
                recv = pltpu.make_async_remote_copy(
                    src_ref=rs_ref.at[s],
                    dst_ref=rs_ref.at[s],
                    send_sem=rs_send_sems.at[s],
                    recv_sem=rs_recv_sems.at[s],
                    device_id=(s,),
                    device_id_type=pl.DeviceIdType.MESH,
                )
                recv.wait_recv()
            reduced = jnp.sum(rs_ref[...].astype(jnp.float32), axis=0)
            for rdma in sends:
                rdma.wait_send()
            return reduced

        part_ref[...] = layer_partial(
            x_ref[...].astype(jnp.bfloat16), win0_ref, wout0_ref
        ).astype(jnp.bfloat16)
        sends = [rs_send(lax.rem(my + off, N_DEV)) for off in range(1, N_DEV)]
        rs_ref[my] = part_ref[pl.ds(my * rows, rows), :]
        reduced = rs_finish(sends)

        for win_ref, wout_ref in [(win1_ref, wout1_ref), (win2_ref, wout2_ref)]:
            ag_ref[my] = reduced.astype(jnp.bfloat16)
            ag_sends = []
            for off in range(1, N_DEV):
                d = lax.rem(my + off, N_DEV)
                rdma = pltpu.make_async_remote_copy(
                    src_ref=ag_ref.at[my],
                    dst_ref=ag_ref.at[my],
                    send_sem=ag_send_sems.at[d],
                    recv_sem=ag_recv_sems.at[my],
                    device_id=(d,),
                    device_id_type=pl.DeviceIdType.MESH,
                )
                rdma.start()
                ag_sends.append(rdma)

            rs_ref[my] = layer_partial(
                ag_ref[my], win_ref, wout_ref
            ).astype(jnp.bfloat16)

            sends = []
            for off in range(1, N_DEV):
                s = lax.rem(my + off, N_DEV)
                recv = pltpu.make_async_remote_copy(
                    src_ref=ag_ref.at[s],
                    dst_ref=ag_ref.at[s],
                    send_sem=ag_send_sems.at[s],
                    recv_sem=ag_recv_sems.at[s],
                    device_id=(s,),
                    device_id_type=pl.DeviceIdType.MESH,
                )
                recv.wait_recv()
                part_ref[pl.ds(s * rows, rows), :] = layer_partial(
                    ag_ref[s], win_ref, wout_ref
                ).astype(jnp.bfloat16)
                sends.append(rs_send(s))
            reduced = rs_finish(sends)
            for rdma in ag_sends:
                rdma.wait_send()

        out_ref[...] = reduced

    out_shape = jax.ShapeDtypeStruct((rows, d_model), jnp.float32)
    vmem = pl.BlockSpec(memory_space=pltpu.VMEM)
    return pl.pallas_call(
        body,
        out_shape=out_shape,
        in_specs=[vmem] * 7,
        out_specs=vmem,
        scratch_shapes=[
            pltpu.VMEM((b, d_model), jnp.bfloat16),
            pltpu.VMEM((N_DEV, rows, d_model), jnp.bfloat16),
            pltpu.VMEM((N_DEV, rows, d_model), jnp.bfloat16),
            pltpu.SemaphoreType.DMA((N_DEV,)),
            pltpu.SemaphoreType.DMA((N_DEV,)),
            pltpu.SemaphoreType.DMA((N_DEV,)),
            pltpu.SemaphoreType.DMA((N_DEV,)),
        ],
    )(x, Win0, Wout0, Win1, Wout1, Win2, Wout2)
